# baseline (speedup 1.0000x reference)
import functools

import jax
import jax.numpy as jnp
from jax.experimental import pallas as pl
from jax.experimental.pallas import tpu as pltpu

TOP_K = 70
TEMPERATURE = 0.1

Q = 4096
N = 100000
D = 512
N_PAD = 100352

BQ = 256
BN = 2048


def _sims_kernel(q_ref, db_ref, out_ref):
    j = pl.program_id(1)
    q = q_ref[...]
    db = db_ref[...]
    qn = q / jnp.maximum(jnp.sqrt(jnp.sum(q * q, axis=-1, keepdims=True)), 1e-12)
    dbn = db / jnp.maximum(jnp.sqrt(jnp.sum(db * db, axis=-1, keepdims=True)), 1e-12)
    sims = jax.lax.dot_general(
        qn, dbn,
        dimension_numbers=(((1,), (1,)), ((), ())),
        preferred_element_type=jnp.float32,
    )
    col = j * BN + jax.lax.broadcasted_iota(jnp.int32, sims.shape, 1)
    out_ref[...] = jnp.where(col < N, sims, -2.0)


@functools.partial(jax.jit, static_argnames=())
def _compute_sims(query, en_db_padded):
    grid = (Q // BQ, N_PAD // BN)
    return pl.pallas_call(
        _sims_kernel,
        grid=grid,
        in_specs=[
            pl.BlockSpec((BQ, D), lambda i, j: (i, 0)),
            pl.BlockSpec((BN, D), lambda i, j: (j, 0)),
        ],
        out_specs=pl.BlockSpec((BQ, BN), lambda i, j: (i, j)),
        out_shape=jax.ShapeDtypeStruct((Q, N_PAD), jnp.float32),
    )(query, en_db_padded)


def kernel(query, en_db, es_db):
    en_db_padded = jnp.pad(en_db, ((0, N_PAD - N), (0, 0)))
    sims = _compute_sims(query, en_db_padded)
    topk_sims, topk_indices = jax.lax.top_k(sims, TOP_K)
    u = jax.random.uniform(jax.random.key(1234), topk_sims.shape,
                           dtype=topk_sims.dtype)
    gumbels = -jnp.log(-jnp.log(u + 1e-10) + 1e-10)
    logits = (topk_sims + gumbels) / TEMPERATURE
    attn_weights = jax.nn.softmax(logits, axis=1)
    topk_es = jnp.take(es_db, topk_indices, axis=0)
    return jnp.einsum('qk,qkd->qd', attn_weights, topk_es)


# speedup vs baseline: 5.4137x; 5.4137x over previous
import functools

import jax
import jax.numpy as jnp
from jax import lax
from jax.experimental import pallas as pl
from jax.experimental.pallas import tpu as pltpu
from jax.experimental.pallas import tpu_sc as plsc

TOP_K = 70
TEMPERATURE = 0.1

Q = 4096
N = 100000
D = 512
N_PAD = 102400
B32 = 32
NB32 = N_PAD // B32
NB256 = N_PAD // 256
BQ = 256
BN = 4096
CAP = 128
KPAD = 72

NCORE = 2
NSUB = 16
NW = NCORE * NSUB
QPW = Q // NW


def _k1_body(q_ref, db_ref, sims_ref, m32_ref, dbn_ref):
    i = pl.program_id(1)

    @pl.when(i == 0)
    def _():
        db = db_ref[...]
        nrm = jnp.maximum(jnp.sqrt(jnp.sum(db * db, axis=-1, keepdims=True)), 1e-12)
        dbn_ref[...] = db / nrm

    q = q_ref[...]
    qn = q / jnp.maximum(jnp.sqrt(jnp.sum(q * q, axis=-1, keepdims=True)), 1e-12)
    s = lax.dot_general(qn, dbn_ref[...],
                        dimension_numbers=(((1,), (1,)), ((), ())),
                        preferred_element_type=jnp.float32)
    j = pl.program_id(0)
    col = j * BN + lax.broadcasted_iota(jnp.int32, s.shape, 1)
    s = jnp.where(col < N, s, -2.0)
    sims_ref[...] = s
    m32_ref[...] = jnp.max(s.reshape(BQ, BN // B32, B32), axis=2)


def _run_k1(query, db_padded):
    return pl.pallas_call(
        _k1_body,
        grid=(N_PAD // BN, Q // BQ),
        in_specs=[
            pl.BlockSpec((BQ, D), lambda j, i: (i, 0)),
            pl.BlockSpec((BN, D), lambda j, i: (j, 0)),
        ],
        out_specs=[
            pl.BlockSpec((BQ, BN), lambda j, i: (i, j)),
            pl.BlockSpec((BQ, BN // B32), lambda j, i: (i, j)),
        ],
        out_shape=[
            jax.ShapeDtypeStruct((Q, N_PAD), jnp.float32),
            jax.ShapeDtypeStruct((Q, NB32), jnp.float32),
        ],
        scratch_shapes=[pltpu.VMEM((BN, D), jnp.float32)],
        compiler_params=pltpu.CompilerParams(vmem_limit_bytes=100 * 2**20),
    )(query, db_padded)


def _k2_body(m_ref, t_ref):
    m32 = m_ref[...]
    M = jnp.max(m32.reshape(BQ, NB256, 8), axis=2)

    def round_body(_, lohi):
        lo, hi = lohi

        def edge_body(k, st):
            nlo, nhi = st
            e = lo + (hi - lo) * ((k.astype(jnp.float32) + 1.0) / 17.0)
            cnt = jnp.sum((M >= e).astype(jnp.float32), axis=1, keepdims=True)
            ok = cnt >= float(TOP_K)
            nlo = jnp.maximum(nlo, jnp.where(ok, e, -4.0))
            nhi = jnp.minimum(nhi, jnp.where(ok, 4.0, e))
            return (nlo, nhi)

        return lax.fori_loop(0, 16, edge_body, (lo, hi))

    lo0 = jnp.full((BQ, 1), -2.0, jnp.float32)
    hi0 = jnp.max(M, axis=1, keepdims=True) + 1e-6
    lo, _ = lax.fori_loop(0, 3, round_body, (lo0, hi0))
    t_ref[...] = jnp.broadcast_to(lo, (BQ, 16))


def _run_k2(m32):
    return pl.pallas_call(
        _k2_body,
        grid=(Q // BQ,),
        in_specs=[pl.BlockSpec((BQ, NB32), lambda i: (i, 0))],
        out_specs=pl.BlockSpec((BQ, 16), lambda i: (i, 0)),
        out_shape=jax.ShapeDtypeStruct((Q, 16), jnp.float32),
        compiler_params=pltpu.CompilerParams(vmem_limit_bytes=100 * 2**20),
    )(m32)


def _s1_body(sims2d, m32_hbm, tb_hbm, simsc_out, idxc_out,
             t_v, m32_v, bid_v, rows_v, sc_v, ic_v, sem):
    c = lax.axis_index("c")
    s = lax.axis_index("s")
    wid = s * NCORE + c
    iota = lax.iota(jnp.int32, 16)

    def per_query(qi, carry):
        q = wid * QPW + qi
        qbase = q * NB32
        pltpu.sync_copy(tb_hbm.at[q], t_v)
        pltpu.sync_copy(m32_hbm.at[q], m32_v)
        tvec = t_v[...]
        qbase_v = jnp.full((16,), qbase, jnp.int32)

        for k in range(CAP // 16):
            bid_v[pl.ds(k * 16, 16)] = qbase_v
            sc_v[pl.ds(k * 16, 16)] = jnp.full((16,), -1e30, jnp.float32)
            ic_v[pl.ds(k * 16, 16)] = jnp.zeros((16,), jnp.int32)

        def scan_body(k, cnt):
            m = m32_v[pl.ds(k * 16, 16)]
            msk = m >= tvec
            cs = plsc.cumsum(msk.astype(jnp.int32))
            pos = cnt + cs - 1
            gid = qbase_v + k * 16 + iota
            plsc.store_scatter(bid_v, [pos], gid, mask=msk & (pos < CAP))
            return cnt + plsc.all_reduce_population_count(msk)

        nb = lax.fori_loop(0, NB32 // 16, scan_body, jnp.zeros((16,), jnp.int32))

        pltpu.async_copy(sims2d.at[bid_v], rows_v, sem).wait()

        def filt_body(rc, cnt2):
            rows16 = iota + rc * 16
            bi = bid_v[pl.ds(rc * 16, 16)]
            bloc = bi - qbase_v
            rvalid = rows16 < nb
            acc = cnt2
            for h in range(B32):
                vals = plsc.load_gather(rows_v, [rows16, jnp.full((16,), h, jnp.int32)])
                msk = (vals >= tvec) & rvalid
                cs = plsc.cumsum(msk.astype(jnp.int32))
                pos = acc + cs - 1
                okm = msk & (pos < CAP)
                plsc.store_scatter(sc_v, [pos], vals, mask=okm)
                plsc.store_scatter(ic_v, [pos], bloc * B32 + h, mask=okm)
                acc = acc + plsc.all_reduce_population_count(msk)
            return acc

        lax.fori_loop(0, CAP // 16, filt_body, jnp.zeros((16,), jnp.int32))

        pltpu.sync_copy(sc_v, simsc_out.at[q])
        pltpu.sync_copy(ic_v, idxc_out.at[q])
        return carry

    lax.fori_loop(0, QPW, per_query, 0)


def _run_s1(sims2d, m32, tb):
    mesh = plsc.VectorSubcoreMesh(core_axis_name="c", subcore_axis_name="s")
    f = functools.partial(
        pl.kernel,
        mesh=mesh,
        compiler_params=pltpu.CompilerParams(needs_layout_passes=False,
                                             use_tc_tiling_on_sc=False),
        out_type=(jax.ShapeDtypeStruct((Q, CAP), jnp.float32),
                  jax.ShapeDtypeStruct((Q, CAP), jnp.int32)),
        scratch_types=[
            pltpu.VMEM((16,), jnp.float32),
            pltpu.VMEM((NB32,), jnp.float32),
            pltpu.VMEM((CAP,), jnp.int32),
            pltpu.VMEM((CAP, B32), jnp.float32),
            pltpu.VMEM((CAP,), jnp.float32),
            pltpu.VMEM((CAP,), jnp.int32),
            pltpu.SemaphoreType.DMA,
        ],
    )(_s1_body)
    return f(sims2d, m32, tb)


def _k3_body(sc_ref, ic_ref, g_ref, w_ref, if_ref):
    x = sc_ref[...]
    idx = ic_ref[...]
    lanes = lax.broadcasted_iota(jnp.int32, x.shape, 1)
    outv = jnp.zeros_like(x)
    outi = jnp.zeros_like(idx)
    for k in range(TOP_K):
        vmax = jnp.max(x, axis=1, keepdims=True)
        ismax = x == vmax
        imin = jnp.min(jnp.where(ismax, idx, jnp.int32(2 ** 30)), axis=1,
                       keepdims=True)
        sel = ismax & (idx == imin)
        outv = jnp.where(lanes == k, vmax, outv)
        outi = jnp.where(lanes == k, imin, outi)
        x = jnp.where(sel, -4.0, x)
    g = g_ref[...]
    logits = jnp.where(lanes < TOP_K, (outv + g) / TEMPERATURE, -1e30)
    m = jnp.max(logits, axis=1, keepdims=True)
    e = jnp.exp(logits - m)
    w_ref[...] = e / jnp.sum(e, axis=1, keepdims=True)
    if_ref[...] = outi


def _run_k3(simsc, idxc, gumbels):
    return pl.pallas_call(
        _k3_body,
        grid=(Q // BQ,),
        in_specs=[
            pl.BlockSpec((BQ, CAP), lambda i: (i, 0)),
            pl.BlockSpec((BQ, CAP), lambda i: (i, 0)),
            pl.BlockSpec((BQ, CAP), lambda i: (i, 0)),
        ],
        out_specs=[
            pl.BlockSpec((BQ, CAP), lambda i: (i, 0)),
            pl.BlockSpec((BQ, CAP), lambda i: (i, 0)),
        ],
        out_shape=[
            jax.ShapeDtypeStruct((Q, CAP), jnp.float32),
            jax.ShapeDtypeStruct((Q, CAP), jnp.int32),
        ],
    )(simsc, idxc, gumbels)


def _s2_body(es_hbm, if_hbm, w_hbm, out_hbm, idx_v, w_v, rows_v, sem):
    c = lax.axis_index("c")
    s = lax.axis_index("s")
    wid = s * NCORE + c

    def per_query(qi, carry):
        q = wid * QPW + qi
        pltpu.sync_copy(if_hbm.at[q, pl.ds(0, KPAD)], idx_v)
        pltpu.sync_copy(w_hbm.at[q, pl.ds(0, KPAD)], w_v)
        pltpu.async_copy(es_hbm.at[idx_v], rows_v, sem).wait()

        nd = D // 16
        acc0 = [jnp.zeros((16,), jnp.float32) for _ in range(nd)]

        def k_body(k, acc):
            wk = plsc.load_gather(w_v, [jnp.full((16,), k, jnp.int32)])
            return [acc[dk] + wk * rows_v[k, pl.ds(dk * 16, 16)]
                    for dk in range(nd)]

        acc = lax.fori_loop(0, TOP_K, k_body, acc0)
        for dk in range(nd):
            rows_v[0, pl.ds(dk * 16, 16)] = acc[dk]
        pltpu.sync_copy(rows_v.at[0], out_hbm.at[q])
        return carry

    lax.fori_loop(0, QPW, per_query, 0)


def _run_s2(es_db, if_arr, w_arr):
    mesh = plsc.VectorSubcoreMesh(core_axis_name="c", subcore_axis_name="s")
    f = functools.partial(
        pl.kernel,
        mesh=mesh,
        compiler_params=pltpu.CompilerParams(needs_layout_passes=False,
                                             use_tc_tiling_on_sc=False),
        out_type=jax.ShapeDtypeStruct((Q, D), jnp.float32),
        scratch_types=[
            pltpu.VMEM((KPAD,), jnp.int32),
            pltpu.VMEM((KPAD,), jnp.float32),
            pltpu.VMEM((KPAD, D), jnp.float32),
            pltpu.SemaphoreType.DMA,
        ],
    )(_s2_body)
    return f(es_db, if_arr, w_arr)


def kernel(query, en_db, es_db):
    db_padded = jnp.pad(en_db, ((0, N_PAD - N), (0, 0)))
    sims, m32 = _run_k1(query, db_padded)
    tb = _run_k2(m32)
    sims2d = sims.reshape(Q * NB32, B32)
    simsc, idxc = _run_s1(sims2d, m32, tb)

    u = jax.random.uniform(jax.random.key(1234), (Q, TOP_K), dtype=jnp.float32)
    gumbels = -jnp.log(-jnp.log(u + 1e-10) + 1e-10)
    gpad = jnp.pad(gumbels, ((0, 0), (0, CAP - TOP_K)))

    w_arr, if_arr = _run_k3(simsc, idxc, gpad)
    return _run_s2(es_db, if_arr, w_arr)
